# baseline (speedup 1.0000x reference)
import dataclasses
import functools

import jax
import jax.numpy as jnp
from jax import lax
from jax.experimental import pallas as pl
from jax.experimental.pallas import tpu as pltpu
from jax.experimental.pallas import tpu_sc as plsc

_B = 4096
_L = 50
_DIM = 300
_V = 100000
_NW = 32
_DMAX = 10
_NC = _B // 16


def kernel(inputs, weight):
    w_t = weight.T
    i_t = inputs.T

    mesh = plsc.VectorSubcoreMesh(core_axis_name="c", subcore_axis_name="s")
    cp = pltpu.CompilerParams()
    if "needs_layout_passes" in pltpu.CompilerParams.__dataclass_fields__:
        cp = dataclasses.replace(cp, needs_layout_passes=False)

    @functools.partial(
        pl.kernel,
        out_type=jax.ShapeDtypeStruct((_L, _DIM, _B), weight.dtype),
        mesh=mesh,
        compiler_params=cp,
        scratch_types=[
            pltpu.VMEM((_V,), jnp.float32),
            pltpu.VMEM((_B,), jnp.int32),
            pltpu.VMEM((_B,), jnp.int32),
            pltpu.VMEM((_B,), jnp.float32),
            pltpu.VMEM((_B,), jnp.float32),
            pltpu.SemaphoreType.DMA,
            pltpu.SemaphoreType.DMA,
            pltpu.SemaphoreType.DMA,
            pltpu.SemaphoreType.DMA,
        ],
    )
    def gather_kernel(w_hbm, i_hbm, o_hbm, row_v,
                      iv0, iv1, ov0, ov1, si0, si1, ss0, ss1):
        sid = lax.axis_index("s")
        cid = lax.axis_index("c")
        wid = sid * 2 + cid
        iv = [iv0, iv1]
        ov = [ov0, ov1]
        si = [si0, si1]
        ss = [ss0, ss1]

        def fire_idx(l, p):
            pltpu.async_copy(i_hbm.at[l], iv[p], si[p])

        def wait_idx(l, p):
            pltpu.make_async_copy(i_hbm.at[l], iv[p], si[p]).wait()

        def fire_store(l, d, p):
            pltpu.async_copy(ov[p], o_hbm.at[l, d], ss[p])

        def wait_store(l, d, p):
            pltpu.make_async_copy(ov[p], o_hbm.at[l, d], ss[p]).wait()

        def compute(p):
            src, dst = iv[p], ov[p]

            @pl.loop(0, _NC, step=32)
            def _(c0):
                vals = []
                for k in range(32):
                    vidx = src[pl.ds((c0 + k) * 16, 16)]
                    vals.append(plsc.load_gather(row_v, [vidx]))
                for k in range(32):
                    dst[pl.ds((c0 + k) * 16, 16)] = vals[k]

        def run_rows(d, l0, nl):
            fire_idx(l0, 0)
            pltpu.sync_copy(w_hbm.at[d], row_v)

            def leg(l, p, first, fire_next):
                wait_idx(l, p)
                if fire_next:
                    fire_idx(l + 1, 1 - p)
                if not first:
                    wait_store(l - 2, d, p)
                compute(p)
                fire_store(l, d, p)

            leg(l0, 0, True, True)
            leg(l0 + 1, 1, True, True)

            npeel = 2 if nl % 2 == 0 else 3

            @pl.loop(0, (nl - 2 - npeel) // 2)
            def _(j):
                l = l0 + 2 + 2 * j
                leg(l, 0, False, True)
                leg(l + 1, 1, False, True)

            if npeel == 3:
                leg(l0 + nl - 3, 0, False, True)
            leg(l0 + nl - 2, nl % 2, False, True)
            leg(l0 + nl - 1, 1 - (nl % 2), False, False)
            wait_store(l0 + nl - 2, d, nl % 2)
            wait_store(l0 + nl - 1, d, 1 - (nl % 2))

        @pl.loop(0, _DIM // _NW)
        def _(i):
            run_rows(wid + _NW * i, 0, _L)

        @pl.when(wid < 2 * (_DIM % _NW))
        def _():
            d = (_DIM // _NW) * _NW + wid // 2
            l0 = (wid % 2) * (_L // 2)
            run_rows(d, l0, _L // 2)

    out = gather_kernel(w_t, i_t)
    return out.transpose(2, 0, 1)
